# baseline (speedup 1.0000x reference)
import jax
import jax.numpy as jnp
from jax import lax
from jax.experimental import pallas as pl
from jax.experimental.pallas import tpu as pltpu
from jax.experimental.pallas import tpu_sc as plsc

N_NODES = 10000
E = 320000
D = 128
PD = D // 2
NW = 32
EW = E // NW
C = 80
NCHUNK = EW // C
NG = C // 16
SD = 2 * PD + D
DD = 2 * PD
UNROLL = 4
INV_SQRT_D = 1.0 / (D ** 0.5)


def _sc_body(srct_hbm, dstt_hbm, src_hbm, dst_hbm, out_hbm,
             src_v, dst_v, sb0, db0, sb1, db1, ob0, ob1, wb,
             sem0, sem1, semo0, semo1, semi):
    wid = lax.axis_index("s") * 2 + lax.axis_index("c")
    lanes = lax.iota(jnp.int32, 16)
    wbase = wid * EW

    pltpu.async_copy(src_hbm.at[pl.ds(wbase, EW)], src_v, semi)
    pltpu.async_copy(dst_hbm.at[pl.ds(wbase, EW)], dst_v, semi).wait()
    pltpu.make_async_copy(src_hbm.at[pl.ds(wbase, EW)], src_v, semi).wait()

    bufs = ((sb0, db0, sem0, ob0, semo0), (sb1, db1, sem1, ob1, semo1))

    def issue(i, b):
        sb, db, sem = bufs[b][:3]
        pltpu.async_copy(srct_hbm.at[src_v.at[pl.ds(i * C, C)]], sb, sem)
        pltpu.async_copy(dstt_hbm.at[dst_v.at[pl.ds(i * C, C)]], db, sem)

    def wait(i, b):
        sb, db, sem = bufs[b][:3]
        pltpu.make_async_copy(srct_hbm.at[src_v.at[pl.ds(i * C, C)]], sb,
                              sem).wait()
        pltpu.make_async_copy(dstt_hbm.at[dst_v.at[pl.ds(i * C, C)]], db,
                              sem).wait()

    def unpk(word):
        return plsc.unpack(plsc.bitcast(word, jnp.bfloat16),
                           format=plsc.PackFormat.INTERLEAVED)

    def compute_store(i, b):
        sb, db, sem, ob, semo = bufs[b]

        @pl.when(i >= 2)
        def _():
            pltpu.make_async_copy(
                ob, out_hbm.at[pl.ds(wbase + (i - 2) * C, C)], semo).wait()
        for g in range(NG):
            ew = lanes + (g * 16)

            def feat(t, acc):
                acc_d, acc_s, fv = acc
                for u in range(UNROLL):
                    fu = fv + u if u else fv
                    fk = jnp.bitwise_or(fu, PD)
                    gs0, gs1 = unpk(plsc.load_gather(sb, [ew, fu]))
                    gd0, gd1 = unpk(plsc.load_gather(db, [ew, fu]))
                    ks0, ks1 = unpk(plsc.load_gather(sb, [ew, fk]))
                    qd0, qd1 = unpk(plsc.load_gather(db, [ew, fk]))
                    d0 = gs0 - gd0
                    d1 = gs1 - gd1
                    acc_d = acc_d + d0 * d0 + d1 * d1
                    acc_s = acc_s + ks0 * qd0 + ks1 * qd1
                return acc_d, acc_s, fv + UNROLL

            zero = jnp.zeros((16,), jnp.float32)
            fv0 = jnp.zeros((16,), jnp.int32)
            acc_d, acc_s, _ = lax.fori_loop(0, PD // UNROLL, feat,
                                            (zero, zero, fv0))

            x = acc_d + 1e-6
            ibits = lax.bitcast_convert_type(x, jnp.int32)
            ibits = 0x5F3759DF - lax.shift_right_logical(ibits, 1)
            y = lax.bitcast_convert_type(ibits, jnp.float32)
            y = y * (1.5 - 0.5 * x * y * y)
            y = y * (1.5 - 0.5 * x * y * y)
            y = y * (1.5 - 0.5 * x * y * y)
            sq = x * y

            dist = jnp.clip(-sq * INV_SQRT_D, -5.0, 5.0)
            score = jnp.clip(acc_s * INV_SQRT_D, -5.0, 5.0)
            w = jnp.exp(dist) * jnp.exp(score)
            wb[...] = w

            def edge(e, c):
                splat = plsc.load_gather(wb, [jnp.full((16,), e, jnp.int32)])
                re = g * 16 + e
                for j in range(8):
                    col = 16 * j
                    v = sb[re, pl.ds(2 * PD + col, 16)]
                    ob[re, pl.ds(col, 16)] = splat * v
                return c

            lax.fori_loop(0, 16, edge, 0)

        pltpu.async_copy(ob, out_hbm.at[pl.ds(wbase + i * C, C)], semo)

    issue(0, 0)

    def pair(p, carry):
        i0 = p * 2
        issue(i0 + 1, 1)
        wait(i0, 0)
        compute_store(i0, 0)

        @pl.when(i0 + 2 < NCHUNK)
        def _():
            issue(i0 + 2, 0)

        wait(i0 + 1, 1)
        compute_store(i0 + 1, 1)
        return carry

    lax.fori_loop(0, NCHUNK // 2, pair, 0)
    wait(NCHUNK - 1, 0)
    compute_store(NCHUNK - 1, 0)
    pltpu.make_async_copy(
        ob1, out_hbm.at[pl.ds(wbase + (NCHUNK - 2) * C, C)], semo1).wait()
    pltpu.make_async_copy(
        ob0, out_hbm.at[pl.ds(wbase + (NCHUNK - 1) * C, C)], semo0).wait()


@jax.jit
def _run(srct, dstt, src, dst):
    mesh = plsc.VectorSubcoreMesh(core_axis_name="c", subcore_axis_name="s")
    f = pl.kernel(
        _sc_body,
        mesh=mesh,
        out_type=jax.ShapeDtypeStruct((E, D), jnp.float32),
        scratch_types=[
            pltpu.VMEM((EW,), jnp.int32),
            pltpu.VMEM((EW,), jnp.int32),
            pltpu.VMEM((C, SD), jnp.float32),
            pltpu.VMEM((C, DD), jnp.float32),
            pltpu.VMEM((C, SD), jnp.float32),
            pltpu.VMEM((C, DD), jnp.float32),
            pltpu.VMEM((C, D), jnp.float32),
            pltpu.VMEM((C, D), jnp.float32),
            pltpu.VMEM((16,), jnp.float32),
            pltpu.SemaphoreType.DMA,
            pltpu.SemaphoreType.DMA,
            pltpu.SemaphoreType.DMA,
            pltpu.SemaphoreType.DMA,
            pltpu.SemaphoreType.DMA,
        ],
        compiler_params=pltpu.CompilerParams(needs_layout_passes=False,
                                             disable_bounds_checks=True),
    )
    return f(srct, dstt, src, dst)


def _pack_bf16(x):
    xb = x.astype(jnp.bfloat16).reshape(N_NODES, PD, 2)
    return lax.bitcast_convert_type(xb, jnp.float32)


def kernel(G_h, K_h, Q_h, V_h, edge_index):
    src = edge_index[0].astype(jnp.int32)
    dst = edge_index[1].astype(jnp.int32)
    gp = _pack_bf16(G_h)
    kp = _pack_bf16(K_h)
    qp = _pack_bf16(Q_h)
    srct = jnp.concatenate([gp, kp, V_h.reshape(N_NODES, D)], axis=1)
    dstt = jnp.concatenate([gp, qp], axis=1)
    out = _run(srct, dstt, src, dst)
    return out.reshape(E, 8, 16)
